# baseline (speedup 1.0000x reference)
import functools

import jax
import jax.numpy as jnp
from jax import lax
from jax.experimental import pallas as pl
from jax.experimental.pallas import tpu as pltpu
from jax.experimental.pallas import tpu_sc as plsc

_B, _L, _D = 32, 2048, 512
_K, _E = 32, 256
_NC, _NS, _LANES = 2, 16, 16


_LS = 1536


def _sc_select_gather_body(feat_hbm, mask_hbm, gath_hbm, sel_hbm, psum_hbm,
                           cnt_hbm, mask_v, sel_v, gidx_v, rows_v, vidx_v,
                           buf_v, acc_v, cnt16_v, sem, sem2):
    b = lax.axis_index("s") * _NC + lax.axis_index("c")
    pltpu.sync_copy(mask_hbm.at[b], mask_v)

    for c in range(_K // _LANES):
        sel_v[pl.ds(c * _LANES, _LANES)] = jnp.full((_LANES,), _L, jnp.int32)

    def chunk_cond(st):
        i, cnt = st
        return jnp.logical_and(i < _L // _LANES, cnt < _K)

    def chunk(st):
        i, cnt = st
        m = mask_v[pl.ds(i * _LANES, _LANES)]
        vmask = m > 0
        ones = vmask.astype(jnp.int32)
        rank = plsc.cumsum(ones) + cnt
        slot = rank - 1
        pos = lax.iota(jnp.int32, _LANES) + i * _LANES
        plsc.store_scatter(sel_v, [slot], pos, mask=vmask & (slot < _K))
        return i + 1, cnt + jnp.sum(ones)

    lax.while_loop(chunk_cond, chunk, (jnp.int32(0), jnp.int32(0)))

    for c in range(_K // _LANES):
        s = sel_v[pl.ds(c * _LANES, _LANES)]
        gidx_v[pl.ds(c * _LANES, _LANES)] = jnp.minimum(s, _L - 1) + b * _L

    pltpu.async_copy(feat_hbm.at[gidx_v], rows_v, sem).wait()
    pltpu.sync_copy(rows_v, gath_hbm.at[b])
    pltpu.sync_copy(sel_v, sel_hbm.at[b])

    def tail_chunk(i, cnt):
        m = mask_v[pl.ds(_LS + i * _LANES, _LANES)]
        vmask = m > 0
        ones = vmask.astype(jnp.int32)
        rank = plsc.cumsum(ones) + cnt
        pos = lax.iota(jnp.int32, _LANES) + (_LS + i * _LANES + b * _L)
        plsc.store_scatter(vidx_v, [rank - 1], pos, mask=vmask)
        return cnt + jnp.sum(ones)

    nv = lax.fori_loop(0, (_L - _LS) // _LANES, tail_chunk, jnp.int32(0))

    for c in range(_D // _LANES):
        acc_v[pl.ds(c * _LANES, _LANES)] = jnp.zeros((_LANES,), jnp.float32)

    def group(i, carry):
        idx16 = vidx_v[pl.ds(i * _LANES, _LANES)]
        pltpu.async_copy(feat_hbm.at[idx16], buf_v, sem2).wait()
        for r in range(_LANES):
            @pl.when(i * _LANES + r < nv)
            def _():
                for c in range(_D // _LANES):
                    plsc.addupdate(acc_v.at[pl.ds(c * _LANES, _LANES)],
                                   buf_v[r, pl.ds(c * _LANES, _LANES)])
        return carry

    ng = (nv + _LANES - 1) // _LANES
    lax.fori_loop(0, ng, group, jnp.int32(0))

    pltpu.sync_copy(acc_v, psum_hbm.at[b])
    cnt16_v[...] = jnp.where(lax.iota(jnp.int32, _LANES) == 0, nv, 0)
    pltpu.sync_copy(cnt16_v, cnt_hbm.at[b])


@functools.cache
def _sc_select_gather():
    return pl.kernel(
        _sc_select_gather_body,
        mesh=plsc.VectorSubcoreMesh(core_axis_name="c", subcore_axis_name="s"),
        compiler_params=pltpu.CompilerParams(needs_layout_passes=False),
        out_type=[
            jax.ShapeDtypeStruct((_B, _K, _D), jnp.float32),
            jax.ShapeDtypeStruct((_B, _K), jnp.int32),
            jax.ShapeDtypeStruct((_B, _D), jnp.float32),
            jax.ShapeDtypeStruct((_B, _LANES), jnp.int32),
        ],
        scratch_types=[
            pltpu.VMEM((_L,), jnp.int32),
            pltpu.VMEM((_K,), jnp.int32),
            pltpu.VMEM((_K,), jnp.int32),
            pltpu.VMEM((_K, _D), jnp.float32),
            pltpu.VMEM((_L - _LS,), jnp.int32),
            pltpu.VMEM((_LANES, _D), jnp.float32),
            pltpu.VMEM((_D,), jnp.float32),
            pltpu.VMEM((_LANES,), jnp.int32),
            pltpu.SemaphoreType.DMA,
            pltpu.SemaphoreType.DMA,
        ],
    )


_RB = 2
_CL2 = 512
_NSTEP = _B // _RB
_NJ = _LS // _CL2


def _tcred_body(mask_ref, feat_ref, sum_ref, cnt_ref):
    j = pl.program_id(1)
    jj = pl.multiple_of(j * _CL2, _CL2)
    mf = (mask_ref[0, :, pl.ds(jj, _CL2)] > 0).astype(jnp.float32)
    x = feat_ref[...]
    part = lax.dot_general(mf, x, (((1,), (1,)), ((0,), (0,))),
                           preferred_element_type=jnp.float32)
    pcnt = jnp.sum(mf, axis=1)[:, None]

    @pl.when(j == 0)
    def _():
        sum_ref[...] = part[None]
        cnt_ref[...] = pcnt[None]

    @pl.when(j > 0)
    def _():
        sum_ref[...] += part[None]
        cnt_ref[...] += pcnt[None]


def _tcred_call(mask3, text_feat):
    return pl.pallas_call(
        _tcred_body,
        grid=(_NSTEP, _NJ),
        in_specs=[
            pl.BlockSpec((1, _RB, _L), lambda i, j: (i, 0, 0)),
            pl.BlockSpec((_RB, _CL2, _D), lambda i, j: (i, j, 0)),
        ],
        out_specs=[
            pl.BlockSpec((1, _RB, _D), lambda i, j: (i, 0, 0)),
            pl.BlockSpec((1, _RB, 1), lambda i, j: (i, 0, 0)),
        ],
        out_shape=[
            jax.ShapeDtypeStruct((_NSTEP, _RB, _D), jnp.float32),
            jax.ShapeDtypeStruct((_NSTEP, _RB, 1), jnp.float32),
        ],
    )(mask3, text_feat)


_KC = 8


def _finish_body(pstc_ref, cnttc_ref, pssc_ref, cntsc_ref, gath_ref, sel_ref,
                 dw_ref, db_ref, tw_ref, tb_ref, base_ref, out_ref):
    psum = pstc_ref[...].reshape(_B, _D) + pssc_ref[...]
    cnt = (cnttc_ref[...].reshape(_B, 1)
           + cntsc_ref[...][:, 0:1].astype(jnp.float32))
    pooled = psum / jnp.maximum(cnt, 1.0)
    g = lax.dot_general(pooled, dw_ref[...], (((1,), (1,)), ((), ())),
                        preferred_element_type=jnp.float32)
    gr = gath_ref[...].reshape(_B * _KC, _D)
    t = lax.dot_general(gr, tw_ref[...], (((1,), (1,)), ((), ())),
                        preferred_element_type=jnp.float32)
    valid = (sel_ref[...] < _L).astype(jnp.float32)
    t3 = (t.reshape(_B, _KC, _E) + tb_ref[...]) * valid
    out_ref[...] = (base_ref[...] + db_ref[...]
                    + g.reshape(_B, _KC, _E) + t3)


def _finish_call(ps_tc, cnt_tc, ps_sc, cnt_sc, gath3, sel3, delta_W,
                 delta_b3, token_W, token_b3, base_tokens):
    return pl.pallas_call(
        _finish_body,
        grid=(_K // _KC,),
        in_specs=[
            pl.BlockSpec((_NSTEP, _RB, _D), lambda j: (0, 0, 0)),
            pl.BlockSpec((_NSTEP, _RB, 1), lambda j: (0, 0, 0)),
            pl.BlockSpec((_B, _D), lambda j: (0, 0)),
            pl.BlockSpec((_B, _LANES), lambda j: (0, 0)),
            pl.BlockSpec((_B, _KC, _D), lambda j: (0, j, 0)),
            pl.BlockSpec((_B, _KC, 1), lambda j: (0, j, 0)),
            pl.BlockSpec((_KC * _E, _D), lambda j: (j, 0)),
            pl.BlockSpec((1, _KC, _E), lambda j: (0, j, 0)),
            pl.BlockSpec((_E, _D), lambda j: (0, 0)),
            pl.BlockSpec((1, 1, _E), lambda j: (0, 0, 0)),
            pl.BlockSpec((1, _KC, _E), lambda j: (0, j, 0)),
        ],
        out_specs=pl.BlockSpec((_B, _KC, _E), lambda j: (0, j, 0)),
        out_shape=jax.ShapeDtypeStruct((_B, _K, _E), jnp.float32),
    )(ps_tc, cnt_tc, ps_sc, cnt_sc, gath3, sel3, delta_W, delta_b3,
      token_W, token_b3, base_tokens)


def kernel(text_feat, attention_mask, base_tokens, delta_W, delta_b,
           token_W, token_b):
    feat_flat = text_feat.reshape(_B * _L, _D)
    gathered, sel, ps_sc, cnt_sc = _sc_select_gather()(feat_flat,
                                                       attention_mask)
    mask3 = attention_mask.reshape(_NSTEP, _RB, _L)
    ps_tc, cnt_tc = _tcred_call(mask3, text_feat)
    out = _finish_call(ps_tc, cnt_tc, ps_sc, cnt_sc, gathered,
                       sel[:, :, None], delta_W, delta_b.reshape(1, _K, _E),
                       token_W, token_b.reshape(1, 1, _E), base_tokens)
    return out


# speedup vs baseline: 1.0105x; 1.0105x over previous
import functools

import jax
import jax.numpy as jnp
from jax import lax
from jax.experimental import pallas as pl
from jax.experimental.pallas import tpu as pltpu
from jax.experimental.pallas import tpu_sc as plsc

_B, _L, _D = 32, 2048, 512
_K, _E = 32, 256
_NC, _NS, _LANES = 2, 16, 16


_LS = 1536
_TCH = 64


def _sc_select_gather_body(feat_hbm, mask_hbm, gath_hbm, sel_hbm, psum_hbm,
                           cnt_hbm, mask_v, sel_v, gidx_v, rows_v,
                           buf_v, acc_v, cnt16_v, sem, sem2):
    b = lax.axis_index("s") * _NC + lax.axis_index("c")
    pltpu.sync_copy(mask_hbm.at[b], mask_v)

    for c in range(_K // _LANES):
        sel_v[pl.ds(c * _LANES, _LANES)] = jnp.full((_LANES,), _L, jnp.int32)

    def chunk_cond(st):
        i, cnt = st
        return jnp.logical_and(i < _L // _LANES, cnt < _K)

    def chunk(st):
        i, cnt = st
        m = mask_v[pl.ds(i * _LANES, _LANES)]
        vmask = m > 0
        ones = vmask.astype(jnp.int32)
        rank = plsc.cumsum(ones) + cnt
        slot = rank - 1
        pos = lax.iota(jnp.int32, _LANES) + i * _LANES
        plsc.store_scatter(sel_v, [slot], pos, mask=vmask & (slot < _K))
        return i + 1, cnt + jnp.sum(ones)

    lax.while_loop(chunk_cond, chunk, (jnp.int32(0), jnp.int32(0)))

    for c in range(_K // _LANES):
        s = sel_v[pl.ds(c * _LANES, _LANES)]
        gidx_v[pl.ds(c * _LANES, _LANES)] = jnp.minimum(s, _L - 1) + b * _L

    pltpu.async_copy(feat_hbm.at[gidx_v], rows_v, sem).wait()
    pltpu.sync_copy(rows_v, gath_hbm.at[b])
    pltpu.sync_copy(sel_v, sel_hbm.at[b])

    for c in range(_D // _LANES):
        acc_v[pl.ds(c * _LANES, _LANES)] = jnp.zeros((_LANES,), jnp.float32)

    def tail_chunk(ci, carry):
        base_row = _LS + ci * _TCH
        pltpu.sync_copy(feat_hbm.at[pl.ds(b * _L + base_row, _TCH)], buf_v)

        def group(g, carry2):
            mvec = mask_v[pl.ds(base_row + g * _LANES, _LANES)]
            for r in range(_LANES):
                @pl.when(mvec[r] > 0)
                def _(g=g, r=r):
                    for c in range(_D // _LANES):
                        plsc.addupdate(
                            acc_v.at[pl.ds(c * _LANES, _LANES)],
                            buf_v[g * _LANES + r, pl.ds(c * _LANES, _LANES)])
            return carry2

        lax.fori_loop(0, _TCH // _LANES, group, jnp.int32(0))
        return carry

    lax.fori_loop(0, (_L - _LS) // _TCH, tail_chunk, jnp.int32(0))

    def cntc(i, cnt):
        m = mask_v[pl.ds(_LS + i * _LANES, _LANES)]
        return cnt + jnp.sum((m > 0).astype(jnp.int32))

    nv = lax.fori_loop(0, (_L - _LS) // _LANES, cntc, jnp.int32(0))

    pltpu.sync_copy(acc_v, psum_hbm.at[b])
    cnt16_v[...] = jnp.where(lax.iota(jnp.int32, _LANES) == 0, nv, 0)
    pltpu.sync_copy(cnt16_v, cnt_hbm.at[b])


@functools.cache
def _sc_select_gather():
    return pl.kernel(
        _sc_select_gather_body,
        mesh=plsc.VectorSubcoreMesh(core_axis_name="c", subcore_axis_name="s"),
        compiler_params=pltpu.CompilerParams(needs_layout_passes=False),
        out_type=[
            jax.ShapeDtypeStruct((_B, _K, _D), jnp.float32),
            jax.ShapeDtypeStruct((_B, _K), jnp.int32),
            jax.ShapeDtypeStruct((_B, _D), jnp.float32),
            jax.ShapeDtypeStruct((_B, _LANES), jnp.int32),
        ],
        scratch_types=[
            pltpu.VMEM((_L,), jnp.int32),
            pltpu.VMEM((_K,), jnp.int32),
            pltpu.VMEM((_K,), jnp.int32),
            pltpu.VMEM((_K, _D), jnp.float32),
            pltpu.VMEM((_TCH, _D), jnp.float32),
            pltpu.VMEM((_D,), jnp.float32),
            pltpu.VMEM((_LANES,), jnp.int32),
            pltpu.SemaphoreType.DMA,
            pltpu.SemaphoreType.DMA,
        ],
    )


_RB = 2
_CL2 = 512
_NSTEP = _B // _RB
_NJ = _LS // _CL2


def _tcred_body(mask_ref, feat_ref, sum_ref, cnt_ref):
    j = pl.program_id(1)
    jj = pl.multiple_of(j * _CL2, _CL2)
    mf = (mask_ref[0, :, pl.ds(jj, _CL2)] > 0).astype(jnp.float32)
    x = feat_ref[...]
    part = lax.dot_general(mf, x, (((1,), (1,)), ((0,), (0,))),
                           preferred_element_type=jnp.float32)
    pcnt = jnp.sum(mf, axis=1)[:, None]

    @pl.when(j == 0)
    def _():
        sum_ref[...] = part[None]
        cnt_ref[...] = pcnt[None]

    @pl.when(j > 0)
    def _():
        sum_ref[...] += part[None]
        cnt_ref[...] += pcnt[None]


def _tcred_call(mask3, text_feat):
    return pl.pallas_call(
        _tcred_body,
        grid=(_NSTEP, _NJ),
        in_specs=[
            pl.BlockSpec((1, _RB, _L), lambda i, j: (i, 0, 0)),
            pl.BlockSpec((_RB, _CL2, _D), lambda i, j: (i, j, 0)),
        ],
        out_specs=[
            pl.BlockSpec((1, _RB, _D), lambda i, j: (i, 0, 0)),
            pl.BlockSpec((1, _RB, 1), lambda i, j: (i, 0, 0)),
        ],
        out_shape=[
            jax.ShapeDtypeStruct((_NSTEP, _RB, _D), jnp.float32),
            jax.ShapeDtypeStruct((_NSTEP, _RB, 1), jnp.float32),
        ],
    )(mask3, text_feat)


_KC = 8


def _finish_body(pstc_ref, cnttc_ref, pssc_ref, cntsc_ref, gath_ref, sel_ref,
                 dw_ref, db_ref, tw_ref, tb_ref, base_ref, out_ref):
    psum = pstc_ref[...].reshape(_B, _D) + pssc_ref[...]
    cnt = (cnttc_ref[...].reshape(_B, 1)
           + cntsc_ref[...][:, 0:1].astype(jnp.float32))
    pooled = psum / jnp.maximum(cnt, 1.0)
    g = lax.dot_general(pooled, dw_ref[...], (((1,), (1,)), ((), ())),
                        preferred_element_type=jnp.float32)
    gr = gath_ref[...].reshape(_B * _KC, _D)
    t = lax.dot_general(gr, tw_ref[...], (((1,), (1,)), ((), ())),
                        preferred_element_type=jnp.float32)
    valid = (sel_ref[...] < _L).astype(jnp.float32)
    t3 = (t.reshape(_B, _KC, _E) + tb_ref[...]) * valid
    out_ref[...] = (base_ref[...] + db_ref[...]
                    + g.reshape(_B, _KC, _E) + t3)


def _finish_call(ps_tc, cnt_tc, ps_sc, cnt_sc, gath3, sel3, delta_W,
                 delta_b3, token_W, token_b3, base_tokens):
    return pl.pallas_call(
        _finish_body,
        grid=(_K // _KC,),
        in_specs=[
            pl.BlockSpec((_NSTEP, _RB, _D), lambda j: (0, 0, 0)),
            pl.BlockSpec((_NSTEP, _RB, 1), lambda j: (0, 0, 0)),
            pl.BlockSpec((_B, _D), lambda j: (0, 0)),
            pl.BlockSpec((_B, _LANES), lambda j: (0, 0)),
            pl.BlockSpec((_B, _KC, _D), lambda j: (0, j, 0)),
            pl.BlockSpec((_B, _KC, 1), lambda j: (0, j, 0)),
            pl.BlockSpec((_KC * _E, _D), lambda j: (j, 0)),
            pl.BlockSpec((1, _KC, _E), lambda j: (0, j, 0)),
            pl.BlockSpec((_E, _D), lambda j: (0, 0)),
            pl.BlockSpec((1, 1, _E), lambda j: (0, 0, 0)),
            pl.BlockSpec((1, _KC, _E), lambda j: (0, j, 0)),
        ],
        out_specs=pl.BlockSpec((_B, _KC, _E), lambda j: (0, j, 0)),
        out_shape=jax.ShapeDtypeStruct((_B, _K, _E), jnp.float32),
    )(ps_tc, cnt_tc, ps_sc, cnt_sc, gath3, sel3, delta_W, delta_b3,
      token_W, token_b3, base_tokens)


def kernel(text_feat, attention_mask, base_tokens, delta_W, delta_b,
           token_W, token_b):
    feat_flat = text_feat.reshape(_B * _L, _D)
    gathered, sel, ps_sc, cnt_sc = _sc_select_gather()(feat_flat,
                                                       attention_mask)
    mask3 = attention_mask.reshape(_NSTEP, _RB, _L)
    ps_tc, cnt_tc = _tcred_call(mask3, text_feat)
    out = _finish_call(ps_tc, cnt_tc, ps_sc, cnt_sc, gathered,
                       sel[:, :, None], delta_W, delta_b.reshape(1, _K, _E),
                       token_W, token_b.reshape(1, 1, _E), base_tokens)
    return out
